# Initial kernel scaffold; baseline (speedup 1.0000x reference)
import jax
import jax.numpy as jnp
from jax.experimental import pallas as pl


def kernel(accessibility_features, edge_index, ln_g, ln_b, fe_w1, fe_b1, fe_w2, fe_b2, gcn1_w, gcn1_b, bn1_g, bn1_b, gat_w, gat_asrc, gat_adst, gat_b, bn2_g, bn2_b, gcn2_w, gcn2_b, bn3_g, bn3_b, pw1, pb1, pw2, pb2, pw3, pb3):
    raise NotImplementedError("write your pallas kernel here")


# speedup vs baseline: 42.5835x; 42.5835x over previous
import functools

import jax
import jax.numpy as jnp
from jax import lax
from jax.experimental import pallas as pl
from jax.experimental.pallas import tpu as pltpu
from jax.experimental.pallas import tpu_sc as plsc

N = 50000
NP8 = N + 8
E = 800000
GRP = 8
EBP = 6272
EPAD = EBP * 128 - E
NC, NSUB = 2, 16
BR = 1000
GRID = N // BR
NGRP_SUB = EBP // GRP // NSUB
DGA, DGB = 25, 24
ROWC, ROWC_LAST = 3200, 2000


def _zero_chunk(s, zer, acc):
    @pl.when(s < NSUB - 1)
    def _():
        pltpu.sync_copy(zer, acc.at[pl.ds(s * ROWC, ROWC)])

    @pl.when(s == NSUB - 1)
    def _():
        pltpu.sync_copy(zer.at[pl.ds(0, ROWC_LAST)],
                        acc.at[pl.ds((NSUB - 1) * ROWC, ROWC_LAST)])


def _write_chunk(s, acc, dst):
    @pl.when(s < NSUB - 1)
    def _():
        pltpu.sync_copy(acc.at[pl.ds(s * ROWC, ROWC)],
                        dst.at[pl.ds(s * ROWC, ROWC)])

    @pl.when(s == NSUB - 1)
    def _():
        pltpu.sync_copy(acc.at[pl.ds((NSUB - 1) * ROWC, ROWC_LAST)],
                        dst.at[pl.ds((NSUB - 1) * ROWC, ROWC_LAST)])


def _deg_body(dst2d, zer1, ones_h, out, acc, didx, ones_v, sem_s):
    c = lax.axis_index("c")
    s = lax.axis_index("s")
    _zero_chunk(s, zer1, acc)
    pltpu.sync_copy(ones_h, ones_v)
    plsc.subcore_barrier()

    w = c * NSUB + s
    base = jnp.where(w < 16, w * DGA * GRP, 16 * DGA * GRP + (w - 16) * DGB * GRP)
    ngrp = jnp.where(w < 16, DGA, DGB)

    def group(g, carry):
        b0 = base + g * GRP
        pltpu.sync_copy(dst2d.at[pl.ds(b0, GRP)], didx)
        sds = []
        for k in range(GRP):
            sds.append(pltpu.async_copy(
                ones_v, acc.at[didx.at[k]], sem_s, add=True))
        for d in sds:
            d.wait()
        return carry

    lax.fori_loop(0, ngrp, group, 0)
    plsc.subcore_barrier()
    _write_chunk(s, acc, out.at[c])


def _deg_call(dst2d, zer1, ones_h):
    return pl.kernel(
        _deg_body,
        out_type=jax.ShapeDtypeStruct((NC, N, 8), jnp.float32),
        mesh=plsc.VectorSubcoreMesh(core_axis_name="c", subcore_axis_name="s",
                                    num_cores=NC, num_subcores=NSUB),
        compiler_params=pltpu.CompilerParams(use_tc_tiling_on_sc=False, needs_layout_passes=False),
        scratch_types=[
            pltpu.VMEM_SHARED((NP8, 8), jnp.float32),
            pltpu.VMEM((GRP, 128), jnp.int32),
            pltpu.VMEM((128, 8), jnp.float32),
            pltpu.SemaphoreType.DMA,
        ],
    )(dst2d, zer1, ones_h)


def _gcn_body(ch, hflat, src2d, dst2d, zer, out, acc, sidx, didx, sidxo,
              rows, sem_g, sem_s):
    c = lax.axis_index("c")
    s = lax.axis_index("s")
    _zero_chunk(s, zer, acc)
    plsc.subcore_barrier()

    base = s * NGRP_SUB * GRP
    coff = c * NP8

    def group(g, carry):
        b0 = base + g * GRP
        pltpu.sync_copy(src2d.at[pl.ds(b0, GRP)], sidx)
        pltpu.sync_copy(dst2d.at[pl.ds(b0, GRP)], didx)
        for half in range(2):
            gds = []
            for j in range(4):
                k = half * 4 + j
                for i in range(8):
                    sidxo[k, pl.ds(i * 16, 16)] = (
                        sidx[k, pl.ds(i * 16, 16)] + coff)
                gds.append(pltpu.async_copy(
                    hflat.at[sidxo.at[k]], rows.at[j], sem_g))
            for d in gds:
                d.wait()
            sds = []
            for j in range(4):
                k = half * 4 + j
                sds.append(pltpu.async_copy(
                    rows.at[j], acc.at[didx.at[k]], sem_s, add=True))
            for d in sds:
                d.wait()
        return carry

    lax.fori_loop(0, NGRP_SUB, group, 0)
    plsc.subcore_barrier()
    _write_chunk(s, acc, out.at[c])


def _gcn_call(ch, hflat, src2d, dst2d, zer):
    return pl.kernel(
        functools.partial(_gcn_body, ch),
        out_type=jax.ShapeDtypeStruct((NC, N, ch), jnp.float32),
        mesh=plsc.VectorSubcoreMesh(core_axis_name="c", subcore_axis_name="s",
                                    num_cores=NC, num_subcores=NSUB),
        compiler_params=pltpu.CompilerParams(use_tc_tiling_on_sc=False, needs_layout_passes=False),
        scratch_types=[
            pltpu.VMEM_SHARED((NP8, ch), jnp.float32),
            pltpu.VMEM((GRP, 128), jnp.int32),
            pltpu.VMEM((GRP, 128), jnp.int32),
            pltpu.VMEM((GRP, 128), jnp.int32),
            pltpu.VMEM((4, 128, ch), jnp.float32),
            pltpu.SemaphoreType.DMA,
            pltpu.SemaphoreType.DMA,
        ],
    )(hflat, src2d, dst2d, zer)


def _gatn_body(hflat, asrcf, adstf, mvec, src2d, dst2d, zer32,
               outn, accn, sidx, didx, sidxo, didxo,
               rows, asrc_r, adst_r, wbuf, mbuf, sem_g, sem_s):
    c = lax.axis_index("c")
    s = lax.axis_index("s")
    _zero_chunk(s, zer32, accn)
    pltpu.sync_copy(mvec, mbuf)
    plsc.subcore_barrier()

    m = mbuf[...]
    base = s * NGRP_SUB * GRP
    coff = c * NP8
    lane0 = lax.iota(jnp.int32, 16)
    zidx = jnp.zeros((16,), jnp.int32)

    def group(g, carry):
        b0 = base + g * GRP
        pltpu.sync_copy(src2d.at[pl.ds(b0, GRP)], sidx)
        pltpu.sync_copy(dst2d.at[pl.ds(b0, GRP)], didx)
        for half in range(4):
            gds = []
            for j in range(2):
                k = half * 2 + j
                for i in range(8):
                    sidxo[k, pl.ds(i * 16, 16)] = (
                        sidx[k, pl.ds(i * 16, 16)] + coff)
                    didxo[k, pl.ds(i * 16, 16)] = (
                        didx[k, pl.ds(i * 16, 16)] + coff)
                gds.append(pltpu.async_copy(
                    hflat.at[sidxo.at[k]], rows.at[j], sem_g))
                gds.append(pltpu.async_copy(
                    asrcf.at[sidxo.at[k]], asrc_r.at[j], sem_g))
                gds.append(pltpu.async_copy(
                    adstf.at[didxo.at[k]], adst_r.at[j], sem_g))
            for d in gds:
                d.wait()
            for j in range(2):
                for i in range(8):
                    lane = lane0 + i * 16
                    a_s = plsc.load_gather(asrc_r.at[j], [lane, zidx])
                    a_d = plsc.load_gather(adst_r.at[j], [lane, zidx])
                    al = a_s + a_d
                    al = jnp.where(al > 0, al, al * jnp.float32(0.2)) - m
                    plsc.store_scatter(wbuf.at[j], [lane, zidx], jnp.exp(al))

                def scale(e, carry2):
                    ws = plsc.load_gather(
                        wbuf.at[j], [jnp.full((16,), e, jnp.int32), zidx])
                    rows[j, e, pl.ds(0, 16)] = rows[j, e, pl.ds(0, 16)] * ws
                    rows[j, e, pl.ds(16, 16)] = rows[j, e, pl.ds(16, 16)] * ws
                    return carry2

                lax.fori_loop(0, 128, scale, 0)
            sds = []
            for j in range(2):
                k = half * 2 + j
                sds.append(pltpu.async_copy(
                    rows.at[j], accn.at[didx.at[k]], sem_s, add=True))
            for d in sds:
                d.wait()
        return carry

    lax.fori_loop(0, NGRP_SUB, group, 0)
    plsc.subcore_barrier()
    _write_chunk(s, accn, outn.at[c])


def _gatn_call(hflat, asrcf, adstf, mvec, src2d, dst2d, zer32):
    return pl.kernel(
        _gatn_body,
        out_type=jax.ShapeDtypeStruct((NC, N, 32), jnp.float32),
        mesh=plsc.VectorSubcoreMesh(core_axis_name="c", subcore_axis_name="s",
                                    num_cores=NC, num_subcores=NSUB),
        compiler_params=pltpu.CompilerParams(use_tc_tiling_on_sc=False, needs_layout_passes=False),
        scratch_types=[
            pltpu.VMEM_SHARED((NP8, 32), jnp.float32),
            pltpu.VMEM((GRP, 128), jnp.int32),
            pltpu.VMEM((GRP, 128), jnp.int32),
            pltpu.VMEM((GRP, 128), jnp.int32),
            pltpu.VMEM((GRP, 128), jnp.int32),
            pltpu.VMEM((2, 128, 32), jnp.float32),
            pltpu.VMEM((2, 128, 8), jnp.float32),
            pltpu.VMEM((2, 128, 8), jnp.float32),
            pltpu.VMEM((2, 128, 8), jnp.float32),
            pltpu.VMEM((16,), jnp.float32),
            pltpu.SemaphoreType.DMA,
            pltpu.SemaphoreType.DMA,
        ],
    )(hflat, asrcf, adstf, mvec, src2d, dst2d, zer32)


def _gatd_body(asrcf, adstf, mvec, src2d, dst2d, zer8,
               outd, accd, sidx, didx, sidxo, didxo,
               asrc_r, adst_r, wbuf, mbuf, sem_g, sem_s):
    c = lax.axis_index("c")
    s = lax.axis_index("s")
    _zero_chunk(s, zer8, accd)
    pltpu.sync_copy(mvec, mbuf)
    plsc.subcore_barrier()

    m = mbuf[...]
    base = s * NGRP_SUB * GRP
    coff = c * NP8
    lane0 = lax.iota(jnp.int32, 16)
    zidx = jnp.zeros((16,), jnp.int32)

    def group(g, carry):
        b0 = base + g * GRP
        pltpu.sync_copy(src2d.at[pl.ds(b0, GRP)], sidx)
        pltpu.sync_copy(dst2d.at[pl.ds(b0, GRP)], didx)
        for half in range(4):
            gds = []
            for j in range(2):
                k = half * 2 + j
                for i in range(8):
                    sidxo[k, pl.ds(i * 16, 16)] = (
                        sidx[k, pl.ds(i * 16, 16)] + coff)
                    didxo[k, pl.ds(i * 16, 16)] = (
                        didx[k, pl.ds(i * 16, 16)] + coff)
                gds.append(pltpu.async_copy(
                    asrcf.at[sidxo.at[k]], asrc_r.at[j], sem_g))
                gds.append(pltpu.async_copy(
                    adstf.at[didxo.at[k]], adst_r.at[j], sem_g))
            for d in gds:
                d.wait()
            for j in range(2):
                for i in range(8):
                    lane = lane0 + i * 16
                    a_s = plsc.load_gather(asrc_r.at[j], [lane, zidx])
                    a_d = plsc.load_gather(adst_r.at[j], [lane, zidx])
                    al = a_s + a_d
                    al = jnp.where(al > 0, al, al * jnp.float32(0.2)) - m
                    plsc.store_scatter(wbuf.at[j], [lane, zidx], jnp.exp(al))
            sds = []
            for j in range(2):
                k = half * 2 + j
                sds.append(pltpu.async_copy(
                    wbuf.at[j], accd.at[didx.at[k]], sem_s, add=True))
            for d in sds:
                d.wait()
        return carry

    lax.fori_loop(0, NGRP_SUB, group, 0)
    plsc.subcore_barrier()
    _write_chunk(s, accd, outd.at[c])


def _gatd_call(asrcf, adstf, mvec, src2d, dst2d, zer8):
    return pl.kernel(
        _gatd_body,
        out_type=jax.ShapeDtypeStruct((NC, N, 8), jnp.float32),
        mesh=plsc.VectorSubcoreMesh(core_axis_name="c", subcore_axis_name="s",
                                    num_cores=NC, num_subcores=NSUB),
        compiler_params=pltpu.CompilerParams(use_tc_tiling_on_sc=False, needs_layout_passes=False),
        scratch_types=[
            pltpu.VMEM_SHARED((NP8, 8), jnp.float32),
            pltpu.VMEM((GRP, 128), jnp.int32),
            pltpu.VMEM((GRP, 128), jnp.int32),
            pltpu.VMEM((GRP, 128), jnp.int32),
            pltpu.VMEM((GRP, 128), jnp.int32),
            pltpu.VMEM((2, 128, 8), jnp.float32),
            pltpu.VMEM((2, 128, 8), jnp.float32),
            pltpu.VMEM((2, 128, 8), jnp.float32),
            pltpu.VMEM((16,), jnp.float32),
            pltpu.SemaphoreType.DMA,
            pltpu.SemaphoreType.DMA,
        ],
    )(asrcf, adstf, mvec, src2d, dst2d, zer8)


def _full(shape):
    return pl.BlockSpec(shape, lambda i: tuple(0 for _ in shape))


def _rows(ch):
    return pl.BlockSpec((BR, ch), lambda i: (i, 0))


def _mm(a, b):
    return jnp.dot(a, b, preferred_element_type=jnp.float32)


def _bn(x, g, b):
    return x * (jnp.float32(1.0) / jnp.sqrt(jnp.float32(1.0 + 1e-5))) * g + b


def _k1_body(x, lng, lnb, w1, b1, w2, b2, gw, x2o, h1o):
    xb = x[...]
    mu = jnp.mean(xb, axis=1, keepdims=True)
    xc = xb - mu
    var = jnp.mean(xc * xc, axis=1, keepdims=True)
    ln = xc / jnp.sqrt(var + 1e-5) * lng[...] + lnb[...]
    a = jnp.maximum(_mm(ln, w1[...]) + b1[...], 0.0)
    a = jnp.maximum(_mm(a, w2[...]) + b2[...], 0.0)
    x2o[...] = a
    h1o[...] = _mm(a, gw[...])


def _k2_body(h1, d0, d1, ha, hb, dv):
    deg = d0[...] + d1[...] + 1.0
    dinv = lax.rsqrt(deg)
    h1s = h1[...] * dinv
    ha[...] = h1s[:, :32]
    hb[...] = h1s[:, 32:]
    dv[...] = dinv


def _k3_body(a1a, a1b, hsa, hsb, dv, g1b, bn1g, bn1b, gatw, aw, bw,
             h2a, h2b, s0, s1, t0, t1, mo, msc):
    i = pl.program_id(0)
    dinv = dv[...]
    acc = jnp.concatenate([a1a[...], a1b[...]], axis=1)
    h1s = jnp.concatenate([hsa[...], hsb[...]], axis=1)
    g1 = (acc + h1s) * dinv + g1b[...]
    x3 = jnp.maximum(_bn(g1, bn1g[...], bn1b[...]), 0.0)
    h2 = _mm(x3, gatw[...])
    h2_0, h2_1 = h2[:, :32], h2[:, 32:]
    h2a[...] = h2_0
    h2b[...] = h2_1
    awv, bwv = aw[...], bw[...]
    as0 = jnp.sum(h2_0 * awv[0, :], axis=1, keepdims=True)
    as1 = jnp.sum(h2_1 * awv[1, :], axis=1, keepdims=True)
    ad0 = jnp.sum(h2_0 * bwv[0, :], axis=1, keepdims=True)
    ad1 = jnp.sum(h2_1 * bwv[1, :], axis=1, keepdims=True)
    s0[...] = as0
    s1[...] = as1
    t0[...] = ad0
    t1[...] = ad1

    @pl.when(i == 0)
    def _():
        msc[0] = jnp.float32(-3e38)
        msc[1] = jnp.float32(-3e38)
        msc[2] = jnp.float32(-3e38)
        msc[3] = jnp.float32(-3e38)

    msc[0] = jnp.maximum(msc[0], jnp.max(as0))
    msc[1] = jnp.maximum(msc[1], jnp.max(as1))
    msc[2] = jnp.maximum(msc[2], jnp.max(ad0))
    msc[3] = jnp.maximum(msc[3], jnp.max(ad1))

    @pl.when(i == GRID - 1)
    def _():
        mm_ = jnp.maximum(msc[0] + msc[2], msc[1] + msc[3])
        mm_ = jnp.where(mm_ > 0, mm_, mm_ * jnp.float32(0.2))
        mo[...] = jnp.full((16,), mm_, jnp.float32)


def _k4_body(n0, n1, d0, d1, h2a, h2b, s0, s1, t0, t1, mv, bn2g, bn2b,
             gatb, g2w, dv, h3a, h3b):
    m = mv[0]
    outs = []
    for nh, dh, hh, sh, th in ((n0, d0, h2a, s0, t0), (n1, d1, h2b, s1, t1)):
        al = sh[...] + th[...]
        al = jnp.where(al > 0, al, al * jnp.float32(0.2)) - m
        ws = jnp.exp(al)
        num = nh[...] + ws * hh[...]
        den = dh[...] + ws
        outs.append(num / (den + 1e-16))
    xg = jnp.concatenate(outs, axis=1) + gatb[...]
    x4 = jnp.maximum(_bn(xg, bn2g[...], bn2b[...]), 0.0)
    h3s = _mm(x4, g2w[...]) * dv[...]
    h3a[...] = h3s[:, :16]
    h3b[...] = h3s[:, 16:]


def _k5_body(a2a, a2b, h3a, h3b, dv, g2b, bn3g, bn3b, w1, b1, w2, b2,
             w3, b3, out):
    acc = jnp.concatenate([a2a[...], a2b[...]], axis=1)
    h3s = jnp.concatenate([h3a[...], h3b[...]], axis=1)
    g2 = (acc + h3s) * dv[...] + g2b[...]
    x5 = jnp.maximum(_bn(g2, bn3g[...], bn3b[...]), 0.0)
    p = jnp.maximum(_mm(x5, w1[...]) + b1[...], 0.0)
    p = jnp.maximum(_mm(p, w2[...]) + b2[...], 0.0)
    z = _mm(p, w3[...]) + b3[...]
    sig = jnp.float32(1.0) / (jnp.float32(1.0) + jnp.exp(-z))
    out[...] = jnp.broadcast_to(sig, (BR, 8))


def _flat_halves(ha, hb):
    z = jnp.zeros((8, ha.shape[1]), jnp.float32)
    return jnp.concatenate([ha, z, hb, z], axis=0)


def kernel(accessibility_features, edge_index, ln_g, ln_b, fe_w1, fe_b1,
           fe_w2, fe_b2, gcn1_w, gcn1_b, bn1_g, bn1_b, gat_w, gat_asrc,
           gat_adst, gat_b, bn2_g, bn2_b, gcn2_w, gcn2_b, bn3_g, bn3_b,
           pw1, pb1, pw2, pb2, pw3, pb3):
    f32 = jnp.float32
    pad_s = jnp.full((EPAD,), N, jnp.int32)
    src2d = jnp.concatenate([edge_index[0], pad_s]).reshape(EBP, 128)
    dst2d = jnp.concatenate([edge_index[1], pad_s]).reshape(EBP, 128)
    zer32 = jnp.zeros((ROWC, 32), f32)
    zer16 = jnp.zeros((ROWC, 16), f32)
    zer8d = jnp.zeros((ROWC, 8), f32)
    ones_h = jnp.ones((128, 8), f32)

    degf = _deg_call(dst2d, zer8d, ones_h)

    x2, h1 = pl.pallas_call(
        _k1_body,
        grid=(GRID,),
        in_specs=[_rows(128), _full((128,)), _full((128,)), _full((128, 64)),
                  _full((64,)), _full((64, 64)), _full((64,)),
                  _full((64, 64))],
        out_specs=[_rows(64), _rows(64)],
        out_shape=[jax.ShapeDtypeStruct((N, 64), f32)] * 2,
    )(accessibility_features, ln_g, ln_b, fe_w1, fe_b1, fe_w2, fe_b2, gcn1_w)

    h1s_a, h1s_b, dinv = pl.pallas_call(
        _k2_body,
        grid=(GRID,),
        in_specs=[_rows(64), _rows(1), _rows(1)],
        out_specs=[_rows(32), _rows(32), _rows(1)],
        out_shape=[jax.ShapeDtypeStruct((N, 32), f32),
                   jax.ShapeDtypeStruct((N, 32), f32),
                   jax.ShapeDtypeStruct((N, 1), f32)],
    )(h1, degf[0, :, 0:1], degf[1, :, 0:1])

    acc1 = _gcn_call(32, _flat_halves(h1s_a, h1s_b), src2d, dst2d, zer32)

    h2a, h2b, s0, s1, t0, t1, mvec = pl.pallas_call(
        _k3_body,
        grid=(GRID,),
        in_specs=[_rows(32), _rows(32), _rows(32), _rows(32), _rows(1),
                  _full((64,)), _full((64,)), _full((64,)), _full((64, 64)),
                  _full((2, 32)), _full((2, 32))],
        out_specs=[_rows(32), _rows(32), _rows(1), _rows(1), _rows(1),
                   _rows(1), pl.BlockSpec((16,), lambda i: (0,))],
        out_shape=[jax.ShapeDtypeStruct((N, 32), f32),
                   jax.ShapeDtypeStruct((N, 32), f32),
                   jax.ShapeDtypeStruct((N, 1), f32),
                   jax.ShapeDtypeStruct((N, 1), f32),
                   jax.ShapeDtypeStruct((N, 1), f32),
                   jax.ShapeDtypeStruct((N, 1), f32),
                   jax.ShapeDtypeStruct((16,), f32)],
        scratch_shapes=[pltpu.SMEM((4,), f32)],
    )(acc1[0], acc1[1], h1s_a, h1s_b, dinv, gcn1_b, bn1_g, bn1_b, gat_w,
      gat_asrc, gat_adst)

    zp8 = jnp.zeros((8, 8), f32)
    pad7 = ((0, 0), (0, 7))
    asrcf = jnp.concatenate([jnp.pad(s0, pad7), zp8,
                             jnp.pad(s1, pad7), zp8], axis=0)
    adstf = jnp.concatenate([jnp.pad(t0, pad7), zp8,
                             jnp.pad(t1, pad7), zp8], axis=0)
    zer8 = jnp.zeros((ROWC, 8), f32)
    h2flat = _flat_halves(h2a, h2b)
    outn = _gatn_call(h2flat, asrcf, adstf, mvec, src2d, dst2d, zer32)
    outd = _gatd_call(asrcf, adstf, mvec, src2d, dst2d, zer8)

    h3a, h3b = pl.pallas_call(
        _k4_body,
        grid=(GRID,),
        in_specs=[_rows(32), _rows(32), _rows(1), _rows(1), _rows(32),
                  _rows(32), _rows(1), _rows(1), _rows(1), _rows(1),
                  _full((16,)), _full((64,)), _full((64,)), _full((64,)),
                  _full((64, 32)), _rows(1)],
        out_specs=[_rows(16), _rows(16)],
        out_shape=[jax.ShapeDtypeStruct((N, 16), f32)] * 2,
    )(outn[0], outn[1], outd[0, :, 0:1], outd[1, :, 0:1],
      h2a, h2b, s0, s1, t0, t1, mvec, bn2_g, bn2_b, gat_b, gcn2_w, dinv)

    acc2 = _gcn_call(16, _flat_halves(h3a, h3b), src2d, dst2d, zer16)

    out = pl.pallas_call(
        _k5_body,
        grid=(GRID,),
        in_specs=[_rows(16), _rows(16), _rows(16), _rows(16), _rows(1),
                  _full((32,)), _full((32,)), _full((32,)),
                  _full((32, 16)), _full((16,)), _full((16, 8)),
                  _full((8,)), _full((8, 1)), _full((1,))],
        out_specs=[_rows(8)],
        out_shape=[jax.ShapeDtypeStruct((N, 8), f32)],
    )(acc2[0], acc2[1], h3a, h3b, dinv, gcn2_b, bn3_g, bn3_b, pw1, pb1,
      pw2, pb2, pw3, pb3)[0]

    return out[:, 0]


# speedup vs baseline: 43.1859x; 1.0141x over previous
import functools

import jax
import jax.numpy as jnp
from jax import lax
from jax.experimental import pallas as pl
from jax.experimental.pallas import tpu as pltpu
from jax.experimental.pallas import tpu_sc as plsc

N = 50000
NP8 = N + 8
E = 800000
GRP = 8
EBP = 6272
EPAD = EBP * 128 - E
NC, NSUB = 2, 16
BR = 1000
GRID = N // BR
NGRP_SUB = EBP // GRP // NSUB
DGA, DGB = 25, 24
ROWC, ROWC_LAST = 3200, 2000


def _zero_chunk(s, zer, acc):
    @pl.when(s < NSUB - 1)
    def _():
        pltpu.sync_copy(zer, acc.at[pl.ds(s * ROWC, ROWC)])

    @pl.when(s == NSUB - 1)
    def _():
        pltpu.sync_copy(zer.at[pl.ds(0, ROWC_LAST)],
                        acc.at[pl.ds((NSUB - 1) * ROWC, ROWC_LAST)])


def _write_chunk(s, acc, dst):
    @pl.when(s < NSUB - 1)
    def _():
        pltpu.sync_copy(acc.at[pl.ds(s * ROWC, ROWC)],
                        dst.at[pl.ds(s * ROWC, ROWC)])

    @pl.when(s == NSUB - 1)
    def _():
        pltpu.sync_copy(acc.at[pl.ds((NSUB - 1) * ROWC, ROWC_LAST)],
                        dst.at[pl.ds((NSUB - 1) * ROWC, ROWC_LAST)])


def _deg_body(dst2d, zer1, ones_h, out, acc, didx, ones_v, sem_s):
    c = lax.axis_index("c")
    s = lax.axis_index("s")
    _zero_chunk(s, zer1, acc)
    pltpu.sync_copy(ones_h, ones_v)
    plsc.subcore_barrier()

    w = c * NSUB + s
    base = jnp.where(w < 16, w * DGA * GRP, 16 * DGA * GRP + (w - 16) * DGB * GRP)
    ngrp = jnp.where(w < 16, DGA, DGB)

    def group(g, carry):
        b0 = base + g * GRP
        pltpu.sync_copy(dst2d.at[pl.ds(b0, GRP)], didx)
        sds = []
        for k in range(GRP):
            sds.append(pltpu.async_copy(
                ones_v, acc.at[didx.at[k]], sem_s, add=True))
        for d in sds:
            d.wait()
        return carry

    lax.fori_loop(0, ngrp, group, 0)
    plsc.subcore_barrier()
    _write_chunk(s, acc, out.at[c])


def _deg_call(dst2d, zer1, ones_h):
    return pl.kernel(
        _deg_body,
        out_type=jax.ShapeDtypeStruct((NC, N, 8), jnp.float32),
        mesh=plsc.VectorSubcoreMesh(core_axis_name="c", subcore_axis_name="s",
                                    num_cores=NC, num_subcores=NSUB),
        compiler_params=pltpu.CompilerParams(use_tc_tiling_on_sc=False, needs_layout_passes=False),
        scratch_types=[
            pltpu.VMEM_SHARED((NP8, 8), jnp.float32),
            pltpu.VMEM((GRP, 128), jnp.int32),
            pltpu.VMEM((128, 8), jnp.float32),
            pltpu.SemaphoreType.DMA,
        ],
    )(dst2d, zer1, ones_h)


def _gcn_body(ch, hflat, src2d, dst2d, zer, out, acc, sidx, didx, sidxo,
              rows, sem_g, sem_s):
    c = lax.axis_index("c")
    s = lax.axis_index("s")
    _zero_chunk(s, zer, acc)
    plsc.subcore_barrier()

    base = s * NGRP_SUB * GRP
    coff = c * NP8

    def group(g, carry):
        b0 = base + g * GRP
        pltpu.sync_copy(src2d.at[pl.ds(b0, GRP)], sidx)
        pltpu.sync_copy(dst2d.at[pl.ds(b0, GRP)], didx)
        for half in range(2):
            gds = []
            for j in range(4):
                k = half * 4 + j
                for i in range(8):
                    sidxo[k, pl.ds(i * 16, 16)] = (
                        sidx[k, pl.ds(i * 16, 16)] + coff)
                gds.append(pltpu.async_copy(
                    hflat.at[sidxo.at[k]], rows.at[j], sem_g))
            for d in gds:
                d.wait()
            sds = []
            for j in range(4):
                k = half * 4 + j
                sds.append(pltpu.async_copy(
                    rows.at[j], acc.at[didx.at[k]], sem_s, add=True))
            for d in sds:
                d.wait()
        return carry

    lax.fori_loop(0, NGRP_SUB, group, 0)
    plsc.subcore_barrier()
    _write_chunk(s, acc, out.at[c])


def _gcn_call(ch, hflat, src2d, dst2d, zer):
    return pl.kernel(
        functools.partial(_gcn_body, ch),
        out_type=jax.ShapeDtypeStruct((NC, N, ch), jnp.float32),
        mesh=plsc.VectorSubcoreMesh(core_axis_name="c", subcore_axis_name="s",
                                    num_cores=NC, num_subcores=NSUB),
        compiler_params=pltpu.CompilerParams(use_tc_tiling_on_sc=False, needs_layout_passes=False),
        scratch_types=[
            pltpu.VMEM_SHARED((NP8, ch), jnp.float32),
            pltpu.VMEM((GRP, 128), jnp.int32),
            pltpu.VMEM((GRP, 128), jnp.int32),
            pltpu.VMEM((GRP, 128), jnp.int32),
            pltpu.VMEM((4, 128, ch), jnp.float32),
            pltpu.SemaphoreType.DMA,
            pltpu.SemaphoreType.DMA,
        ],
    )(hflat, src2d, dst2d, zer)


def _gatn_body(hflat, asrcf, adstf, mvec, src2d, dst2d, zer32,
               outn, accn, sidx, didx, sidxo, didxo,
               rows, asrc_r, adst_r, wbuf, mbuf, sem_g, sem_s):
    c = lax.axis_index("c")
    s = lax.axis_index("s")
    _zero_chunk(s, zer32, accn)
    pltpu.sync_copy(mvec, mbuf)
    plsc.subcore_barrier()

    m = mbuf[...]
    base = s * NGRP_SUB * GRP
    coff = c * NP8
    lane0 = lax.iota(jnp.int32, 16)
    zidx = jnp.zeros((16,), jnp.int32)

    def group(g, carry):
        b0 = base + g * GRP
        pltpu.sync_copy(src2d.at[pl.ds(b0, GRP)], sidx)
        pltpu.sync_copy(dst2d.at[pl.ds(b0, GRP)], didx)
        for half in range(4):
            gds = []
            for j in range(2):
                k = half * 2 + j
                for i in range(8):
                    sidxo[k, pl.ds(i * 16, 16)] = (
                        sidx[k, pl.ds(i * 16, 16)] + coff)
                    didxo[k, pl.ds(i * 16, 16)] = (
                        didx[k, pl.ds(i * 16, 16)] + coff)
                gds.append(pltpu.async_copy(
                    hflat.at[sidxo.at[k]], rows.at[j], sem_g))
                gds.append(pltpu.async_copy(
                    asrcf.at[sidxo.at[k]], asrc_r.at[j], sem_g))
                gds.append(pltpu.async_copy(
                    adstf.at[didxo.at[k]], adst_r.at[j], sem_g))
            for d in gds:
                d.wait()
            for j in range(2):
                for i in range(8):
                    lane = lane0 + i * 16
                    a_s = plsc.load_gather(asrc_r.at[j], [lane, zidx])
                    a_d = plsc.load_gather(adst_r.at[j], [lane, zidx])
                    al = a_s + a_d
                    al = jnp.where(al > 0, al, al * jnp.float32(0.2)) - m
                    plsc.store_scatter(wbuf.at[j], [lane, zidx], jnp.exp(al))

                def scale(e, carry2):
                    ws = plsc.load_gather(
                        wbuf.at[j], [jnp.full((16,), e, jnp.int32), zidx])
                    rows[j, e, pl.ds(0, 16)] = rows[j, e, pl.ds(0, 16)] * ws
                    rows[j, e, pl.ds(16, 16)] = rows[j, e, pl.ds(16, 16)] * ws
                    return carry2

                lax.fori_loop(0, 128, scale, 0, unroll=8)
            sds = []
            for j in range(2):
                k = half * 2 + j
                sds.append(pltpu.async_copy(
                    rows.at[j], accn.at[didx.at[k]], sem_s, add=True))
            for d in sds:
                d.wait()
        return carry

    lax.fori_loop(0, NGRP_SUB, group, 0)
    plsc.subcore_barrier()
    _write_chunk(s, accn, outn.at[c])


def _gatn_call(hflat, asrcf, adstf, mvec, src2d, dst2d, zer32):
    return pl.kernel(
        _gatn_body,
        out_type=jax.ShapeDtypeStruct((NC, N, 32), jnp.float32),
        mesh=plsc.VectorSubcoreMesh(core_axis_name="c", subcore_axis_name="s",
                                    num_cores=NC, num_subcores=NSUB),
        compiler_params=pltpu.CompilerParams(use_tc_tiling_on_sc=False, needs_layout_passes=False),
        scratch_types=[
            pltpu.VMEM_SHARED((NP8, 32), jnp.float32),
            pltpu.VMEM((GRP, 128), jnp.int32),
            pltpu.VMEM((GRP, 128), jnp.int32),
            pltpu.VMEM((GRP, 128), jnp.int32),
            pltpu.VMEM((GRP, 128), jnp.int32),
            pltpu.VMEM((2, 128, 32), jnp.float32),
            pltpu.VMEM((2, 128, 8), jnp.float32),
            pltpu.VMEM((2, 128, 8), jnp.float32),
            pltpu.VMEM((2, 128, 8), jnp.float32),
            pltpu.VMEM((16,), jnp.float32),
            pltpu.SemaphoreType.DMA,
            pltpu.SemaphoreType.DMA,
        ],
    )(hflat, asrcf, adstf, mvec, src2d, dst2d, zer32)


def _gatd_body(asrcf, adstf, mvec, src2d, dst2d, zer8,
               outd, accd, sidx, didx, sidxo, didxo,
               asrc_r, adst_r, wbuf, mbuf, sem_g, sem_s):
    c = lax.axis_index("c")
    s = lax.axis_index("s")
    _zero_chunk(s, zer8, accd)
    pltpu.sync_copy(mvec, mbuf)
    plsc.subcore_barrier()

    m = mbuf[...]
    base = s * NGRP_SUB * GRP
    coff = c * NP8
    lane0 = lax.iota(jnp.int32, 16)
    zidx = jnp.zeros((16,), jnp.int32)

    def group(g, carry):
        b0 = base + g * GRP
        pltpu.sync_copy(src2d.at[pl.ds(b0, GRP)], sidx)
        pltpu.sync_copy(dst2d.at[pl.ds(b0, GRP)], didx)
        for half in range(4):
            gds = []
            for j in range(2):
                k = half * 2 + j
                for i in range(8):
                    sidxo[k, pl.ds(i * 16, 16)] = (
                        sidx[k, pl.ds(i * 16, 16)] + coff)
                    didxo[k, pl.ds(i * 16, 16)] = (
                        didx[k, pl.ds(i * 16, 16)] + coff)
                gds.append(pltpu.async_copy(
                    asrcf.at[sidxo.at[k]], asrc_r.at[j], sem_g))
                gds.append(pltpu.async_copy(
                    adstf.at[didxo.at[k]], adst_r.at[j], sem_g))
            for d in gds:
                d.wait()
            for j in range(2):
                for i in range(8):
                    lane = lane0 + i * 16
                    a_s = plsc.load_gather(asrc_r.at[j], [lane, zidx])
                    a_d = plsc.load_gather(adst_r.at[j], [lane, zidx])
                    al = a_s + a_d
                    al = jnp.where(al > 0, al, al * jnp.float32(0.2)) - m
                    plsc.store_scatter(wbuf.at[j], [lane, zidx], jnp.exp(al))
            sds = []
            for j in range(2):
                k = half * 2 + j
                sds.append(pltpu.async_copy(
                    wbuf.at[j], accd.at[didx.at[k]], sem_s, add=True))
            for d in sds:
                d.wait()
        return carry

    lax.fori_loop(0, NGRP_SUB, group, 0)
    plsc.subcore_barrier()
    _write_chunk(s, accd, outd.at[c])


def _gatd_call(asrcf, adstf, mvec, src2d, dst2d, zer8):
    return pl.kernel(
        _gatd_body,
        out_type=jax.ShapeDtypeStruct((NC, N, 8), jnp.float32),
        mesh=plsc.VectorSubcoreMesh(core_axis_name="c", subcore_axis_name="s",
                                    num_cores=NC, num_subcores=NSUB),
        compiler_params=pltpu.CompilerParams(use_tc_tiling_on_sc=False, needs_layout_passes=False),
        scratch_types=[
            pltpu.VMEM_SHARED((NP8, 8), jnp.float32),
            pltpu.VMEM((GRP, 128), jnp.int32),
            pltpu.VMEM((GRP, 128), jnp.int32),
            pltpu.VMEM((GRP, 128), jnp.int32),
            pltpu.VMEM((GRP, 128), jnp.int32),
            pltpu.VMEM((2, 128, 8), jnp.float32),
            pltpu.VMEM((2, 128, 8), jnp.float32),
            pltpu.VMEM((2, 128, 8), jnp.float32),
            pltpu.VMEM((16,), jnp.float32),
            pltpu.SemaphoreType.DMA,
            pltpu.SemaphoreType.DMA,
        ],
    )(asrcf, adstf, mvec, src2d, dst2d, zer8)


def _full(shape):
    return pl.BlockSpec(shape, lambda i: tuple(0 for _ in shape))


def _rows(ch):
    return pl.BlockSpec((BR, ch), lambda i: (i, 0))


def _mm(a, b):
    return jnp.dot(a, b, preferred_element_type=jnp.float32)


def _bn(x, g, b):
    return x * (jnp.float32(1.0) / jnp.sqrt(jnp.float32(1.0 + 1e-5))) * g + b


def _k1_body(x, lng, lnb, w1, b1, w2, b2, gw, x2o, h1o):
    xb = x[...]
    mu = jnp.mean(xb, axis=1, keepdims=True)
    xc = xb - mu
    var = jnp.mean(xc * xc, axis=1, keepdims=True)
    ln = xc / jnp.sqrt(var + 1e-5) * lng[...] + lnb[...]
    a = jnp.maximum(_mm(ln, w1[...]) + b1[...], 0.0)
    a = jnp.maximum(_mm(a, w2[...]) + b2[...], 0.0)
    x2o[...] = a
    h1o[...] = _mm(a, gw[...])


def _k2_body(h1, d0, d1, ha, hb, dv):
    deg = d0[...] + d1[...] + 1.0
    dinv = lax.rsqrt(deg)
    h1s = h1[...] * dinv
    ha[...] = h1s[:, :32]
    hb[...] = h1s[:, 32:]
    dv[...] = dinv


def _k3_body(a1a, a1b, hsa, hsb, dv, g1b, bn1g, bn1b, gatw, aw, bw,
             h2a, h2b, s0, s1, t0, t1, mo, msc):
    i = pl.program_id(0)
    dinv = dv[...]
    acc = jnp.concatenate([a1a[...], a1b[...]], axis=1)
    h1s = jnp.concatenate([hsa[...], hsb[...]], axis=1)
    g1 = (acc + h1s) * dinv + g1b[...]
    x3 = jnp.maximum(_bn(g1, bn1g[...], bn1b[...]), 0.0)
    h2 = _mm(x3, gatw[...])
    h2_0, h2_1 = h2[:, :32], h2[:, 32:]
    h2a[...] = h2_0
    h2b[...] = h2_1
    awv, bwv = aw[...], bw[...]
    as0 = jnp.sum(h2_0 * awv[0, :], axis=1, keepdims=True)
    as1 = jnp.sum(h2_1 * awv[1, :], axis=1, keepdims=True)
    ad0 = jnp.sum(h2_0 * bwv[0, :], axis=1, keepdims=True)
    ad1 = jnp.sum(h2_1 * bwv[1, :], axis=1, keepdims=True)
    s0[...] = as0
    s1[...] = as1
    t0[...] = ad0
    t1[...] = ad1

    @pl.when(i == 0)
    def _():
        msc[0] = jnp.float32(-3e38)
        msc[1] = jnp.float32(-3e38)
        msc[2] = jnp.float32(-3e38)
        msc[3] = jnp.float32(-3e38)

    msc[0] = jnp.maximum(msc[0], jnp.max(as0))
    msc[1] = jnp.maximum(msc[1], jnp.max(as1))
    msc[2] = jnp.maximum(msc[2], jnp.max(ad0))
    msc[3] = jnp.maximum(msc[3], jnp.max(ad1))

    @pl.when(i == GRID - 1)
    def _():
        mm_ = jnp.maximum(msc[0] + msc[2], msc[1] + msc[3])
        mm_ = jnp.where(mm_ > 0, mm_, mm_ * jnp.float32(0.2))
        mo[...] = jnp.full((16,), mm_, jnp.float32)


def _k4_body(n0, n1, d0, d1, h2a, h2b, s0, s1, t0, t1, mv, bn2g, bn2b,
             gatb, g2w, dv, h3a, h3b):
    m = mv[0]
    outs = []
    for nh, dh, hh, sh, th in ((n0, d0, h2a, s0, t0), (n1, d1, h2b, s1, t1)):
        al = sh[...] + th[...]
        al = jnp.where(al > 0, al, al * jnp.float32(0.2)) - m
        ws = jnp.exp(al)
        num = nh[...] + ws * hh[...]
        den = dh[...] + ws
        outs.append(num / (den + 1e-16))
    xg = jnp.concatenate(outs, axis=1) + gatb[...]
    x4 = jnp.maximum(_bn(xg, bn2g[...], bn2b[...]), 0.0)
    h3s = _mm(x4, g2w[...]) * dv[...]
    h3a[...] = h3s[:, :16]
    h3b[...] = h3s[:, 16:]


def _k5_body(a2a, a2b, h3a, h3b, dv, g2b, bn3g, bn3b, w1, b1, w2, b2,
             w3, b3, out):
    acc = jnp.concatenate([a2a[...], a2b[...]], axis=1)
    h3s = jnp.concatenate([h3a[...], h3b[...]], axis=1)
    g2 = (acc + h3s) * dv[...] + g2b[...]
    x5 = jnp.maximum(_bn(g2, bn3g[...], bn3b[...]), 0.0)
    p = jnp.maximum(_mm(x5, w1[...]) + b1[...], 0.0)
    p = jnp.maximum(_mm(p, w2[...]) + b2[...], 0.0)
    z = _mm(p, w3[...]) + b3[...]
    sig = jnp.float32(1.0) / (jnp.float32(1.0) + jnp.exp(-z))
    out[...] = jnp.broadcast_to(sig, (BR, 8))


def _flat_halves(ha, hb):
    z = jnp.zeros((8, ha.shape[1]), jnp.float32)
    return jnp.concatenate([ha, z, hb, z], axis=0)


def kernel(accessibility_features, edge_index, ln_g, ln_b, fe_w1, fe_b1,
           fe_w2, fe_b2, gcn1_w, gcn1_b, bn1_g, bn1_b, gat_w, gat_asrc,
           gat_adst, gat_b, bn2_g, bn2_b, gcn2_w, gcn2_b, bn3_g, bn3_b,
           pw1, pb1, pw2, pb2, pw3, pb3):
    f32 = jnp.float32
    pad_s = jnp.full((EPAD,), N, jnp.int32)
    src2d = jnp.concatenate([edge_index[0], pad_s]).reshape(EBP, 128)
    dst2d = jnp.concatenate([edge_index[1], pad_s]).reshape(EBP, 128)
    zer32 = jnp.zeros((ROWC, 32), f32)
    zer16 = jnp.zeros((ROWC, 16), f32)
    zer8d = jnp.zeros((ROWC, 8), f32)
    ones_h = jnp.ones((128, 8), f32)

    degf = _deg_call(dst2d, zer8d, ones_h)

    x2, h1 = pl.pallas_call(
        _k1_body,
        grid=(GRID,),
        in_specs=[_rows(128), _full((128,)), _full((128,)), _full((128, 64)),
                  _full((64,)), _full((64, 64)), _full((64,)),
                  _full((64, 64))],
        out_specs=[_rows(64), _rows(64)],
        out_shape=[jax.ShapeDtypeStruct((N, 64), f32)] * 2,
    )(accessibility_features, ln_g, ln_b, fe_w1, fe_b1, fe_w2, fe_b2, gcn1_w)

    h1s_a, h1s_b, dinv = pl.pallas_call(
        _k2_body,
        grid=(GRID,),
        in_specs=[_rows(64), _rows(1), _rows(1)],
        out_specs=[_rows(32), _rows(32), _rows(1)],
        out_shape=[jax.ShapeDtypeStruct((N, 32), f32),
                   jax.ShapeDtypeStruct((N, 32), f32),
                   jax.ShapeDtypeStruct((N, 1), f32)],
    )(h1, degf[0, :, 0:1], degf[1, :, 0:1])

    acc1 = _gcn_call(32, _flat_halves(h1s_a, h1s_b), src2d, dst2d, zer32)

    h2a, h2b, s0, s1, t0, t1, mvec = pl.pallas_call(
        _k3_body,
        grid=(GRID,),
        in_specs=[_rows(32), _rows(32), _rows(32), _rows(32), _rows(1),
                  _full((64,)), _full((64,)), _full((64,)), _full((64, 64)),
                  _full((2, 32)), _full((2, 32))],
        out_specs=[_rows(32), _rows(32), _rows(1), _rows(1), _rows(1),
                   _rows(1), pl.BlockSpec((16,), lambda i: (0,))],
        out_shape=[jax.ShapeDtypeStruct((N, 32), f32),
                   jax.ShapeDtypeStruct((N, 32), f32),
                   jax.ShapeDtypeStruct((N, 1), f32),
                   jax.ShapeDtypeStruct((N, 1), f32),
                   jax.ShapeDtypeStruct((N, 1), f32),
                   jax.ShapeDtypeStruct((N, 1), f32),
                   jax.ShapeDtypeStruct((16,), f32)],
        scratch_shapes=[pltpu.SMEM((4,), f32)],
    )(acc1[0], acc1[1], h1s_a, h1s_b, dinv, gcn1_b, bn1_g, bn1_b, gat_w,
      gat_asrc, gat_adst)

    zp8 = jnp.zeros((8, 8), f32)
    pad7 = ((0, 0), (0, 7))
    asrcf = jnp.concatenate([jnp.pad(s0, pad7), zp8,
                             jnp.pad(s1, pad7), zp8], axis=0)
    adstf = jnp.concatenate([jnp.pad(t0, pad7), zp8,
                             jnp.pad(t1, pad7), zp8], axis=0)
    zer8 = jnp.zeros((ROWC, 8), f32)
    h2flat = _flat_halves(h2a, h2b)
    outn = _gatn_call(h2flat, asrcf, adstf, mvec, src2d, dst2d, zer32)
    outd = _gatd_call(asrcf, adstf, mvec, src2d, dst2d, zer8)

    h3a, h3b = pl.pallas_call(
        _k4_body,
        grid=(GRID,),
        in_specs=[_rows(32), _rows(32), _rows(1), _rows(1), _rows(32),
                  _rows(32), _rows(1), _rows(1), _rows(1), _rows(1),
                  _full((16,)), _full((64,)), _full((64,)), _full((64,)),
                  _full((64, 32)), _rows(1)],
        out_specs=[_rows(16), _rows(16)],
        out_shape=[jax.ShapeDtypeStruct((N, 16), f32)] * 2,
    )(outn[0], outn[1], outd[0, :, 0:1], outd[1, :, 0:1],
      h2a, h2b, s0, s1, t0, t1, mvec, bn2_g, bn2_b, gat_b, gcn2_w, dinv)

    acc2 = _gcn_call(16, _flat_halves(h3a, h3b), src2d, dst2d, zer16)

    out = pl.pallas_call(
        _k5_body,
        grid=(GRID,),
        in_specs=[_rows(16), _rows(16), _rows(16), _rows(16), _rows(1),
                  _full((32,)), _full((32,)), _full((32,)),
                  _full((32, 16)), _full((16,)), _full((16, 8)),
                  _full((8,)), _full((8, 1)), _full((1,))],
        out_specs=[_rows(8)],
        out_shape=[jax.ShapeDtypeStruct((N, 8), f32)],
    )(acc2[0], acc2[1], h3a, h3b, dinv, gcn2_b, bn3_g, bn3_b, pw1, pb1,
      pw2, pb2, pw3, pb3)[0]

    return out[:, 0]


# speedup vs baseline: 47.0185x; 1.0887x over previous
import functools

import jax
import jax.numpy as jnp
from jax import lax
from jax.experimental import pallas as pl
from jax.experimental.pallas import tpu as pltpu
from jax.experimental.pallas import tpu_sc as plsc

N = 50000
NP8 = N + 8
E = 800000
GRP = 8
EBP = 6272
EPAD = EBP * 128 - E
NC, NSUB = 2, 16
BR = 1000
GRID = N // BR
NGRP_SUB = EBP // GRP // NSUB
DGA, DGB = 25, 24
ROWC, ROWC_LAST = 3200, 2000


def _zero_chunk(s, zer, acc):
    @pl.when(s < NSUB - 1)
    def _():
        pltpu.sync_copy(zer, acc.at[pl.ds(s * ROWC, ROWC)])

    @pl.when(s == NSUB - 1)
    def _():
        pltpu.sync_copy(zer.at[pl.ds(0, ROWC_LAST)],
                        acc.at[pl.ds((NSUB - 1) * ROWC, ROWC_LAST)])


def _write_chunk(s, acc, dst):
    @pl.when(s < NSUB - 1)
    def _():
        pltpu.sync_copy(acc.at[pl.ds(s * ROWC, ROWC)],
                        dst.at[pl.ds(s * ROWC, ROWC)])

    @pl.when(s == NSUB - 1)
    def _():
        pltpu.sync_copy(acc.at[pl.ds((NSUB - 1) * ROWC, ROWC_LAST)],
                        dst.at[pl.ds((NSUB - 1) * ROWC, ROWC_LAST)])


def _deg_body(dst2d, zer1, ones_h, out, acc, didx, ones_v, sem_s):
    c = lax.axis_index("c")
    s = lax.axis_index("s")
    _zero_chunk(s, zer1, acc)
    pltpu.sync_copy(ones_h, ones_v)
    plsc.subcore_barrier()

    w = c * NSUB + s
    base = jnp.where(w < 16, w * DGA * GRP, 16 * DGA * GRP + (w - 16) * DGB * GRP)
    ngrp = jnp.where(w < 16, DGA, DGB)

    def group(g, carry):
        b0 = base + g * GRP
        pltpu.sync_copy(dst2d.at[pl.ds(b0, GRP)], didx)
        sds = []
        for k in range(GRP):
            sds.append(pltpu.async_copy(
                ones_v, acc.at[didx.at[k]], sem_s, add=True))
        for d in sds:
            d.wait()
        return carry

    lax.fori_loop(0, ngrp, group, 0)
    plsc.subcore_barrier()
    _write_chunk(s, acc, out.at[c])


def _deg_call(dst2d, zer1, ones_h):
    return pl.kernel(
        _deg_body,
        out_type=jax.ShapeDtypeStruct((NC, N, 8), jnp.float32),
        mesh=plsc.VectorSubcoreMesh(core_axis_name="c", subcore_axis_name="s",
                                    num_cores=NC, num_subcores=NSUB),
        compiler_params=pltpu.CompilerParams(use_tc_tiling_on_sc=False, needs_layout_passes=False),
        scratch_types=[
            pltpu.VMEM_SHARED((NP8, 8), jnp.float32),
            pltpu.VMEM((GRP, 128), jnp.int32),
            pltpu.VMEM((128, 8), jnp.float32),
            pltpu.SemaphoreType.DMA,
        ],
    )(dst2d, zer1, ones_h)


def _gcn_body(ch, hflat, src2d, dst2d, zer, out, acc, sidx, didx, sidxo,
              rows, sem_g, sem_s):
    c = lax.axis_index("c")
    s = lax.axis_index("s")
    _zero_chunk(s, zer, acc)
    plsc.subcore_barrier()

    base = s * NGRP_SUB * GRP
    coff = c * NP8

    def group(g, carry):
        b0 = base + g * GRP
        pltpu.sync_copy(src2d.at[pl.ds(b0, GRP)], sidx)
        pltpu.sync_copy(dst2d.at[pl.ds(b0, GRP)], didx)
        for half in range(2):
            gds = []
            for j in range(4):
                k = half * 4 + j
                for i in range(8):
                    sidxo[k, pl.ds(i * 16, 16)] = (
                        sidx[k, pl.ds(i * 16, 16)] + coff)
                gds.append(pltpu.async_copy(
                    hflat.at[sidxo.at[k]], rows.at[j], sem_g))
            for d in gds:
                d.wait()
            sds = []
            for j in range(4):
                k = half * 4 + j
                sds.append(pltpu.async_copy(
                    rows.at[j], acc.at[didx.at[k]], sem_s, add=True))
            for d in sds:
                d.wait()
        return carry

    lax.fori_loop(0, NGRP_SUB, group, 0)
    plsc.subcore_barrier()
    _write_chunk(s, acc, out.at[c])


def _gcn_call(ch, hflat, src2d, dst2d, zer):
    return pl.kernel(
        functools.partial(_gcn_body, ch),
        out_type=jax.ShapeDtypeStruct((NC, N, ch), jnp.float32),
        mesh=plsc.VectorSubcoreMesh(core_axis_name="c", subcore_axis_name="s",
                                    num_cores=NC, num_subcores=NSUB),
        compiler_params=pltpu.CompilerParams(use_tc_tiling_on_sc=False, needs_layout_passes=False),
        scratch_types=[
            pltpu.VMEM_SHARED((NP8, ch), jnp.float32),
            pltpu.VMEM((GRP, 128), jnp.int32),
            pltpu.VMEM((GRP, 128), jnp.int32),
            pltpu.VMEM((GRP, 128), jnp.int32),
            pltpu.VMEM((4, 128, ch), jnp.float32),
            pltpu.SemaphoreType.DMA,
            pltpu.SemaphoreType.DMA,
        ],
    )(hflat, src2d, dst2d, zer)


def _gatn_body(hflat, asrcf, adstf, mvec, src2d, dst2d, zer32,
               outn, wout, accn, sidx, didx, sidxo, didxo,
               rows, asrc_r, adst_r, wbuf, mbuf, sem_g, sem_s, sem_w):
    c = lax.axis_index("c")
    s = lax.axis_index("s")
    _zero_chunk(s, zer32, accn)
    pltpu.sync_copy(mvec, mbuf)
    plsc.subcore_barrier()

    m = mbuf[...]
    base = s * NGRP_SUB * GRP
    coff = c * NP8
    lane0 = lax.iota(jnp.int32, 16)
    zidx = jnp.zeros((16,), jnp.int32)

    def group(g, carry):
        b0 = base + g * GRP
        pltpu.sync_copy(src2d.at[pl.ds(b0, GRP)], sidx)
        pltpu.sync_copy(dst2d.at[pl.ds(b0, GRP)], didx)
        for half in range(4):
            gds = []
            for j in range(2):
                k = half * 2 + j
                for i in range(8):
                    sidxo[k, pl.ds(i * 16, 16)] = (
                        sidx[k, pl.ds(i * 16, 16)] + coff)
                    didxo[k, pl.ds(i * 16, 16)] = (
                        didx[k, pl.ds(i * 16, 16)] + coff)
                gds.append(pltpu.async_copy(
                    hflat.at[sidxo.at[k]], rows.at[j], sem_g))
                gds.append(pltpu.async_copy(
                    asrcf.at[sidxo.at[k]], asrc_r.at[j], sem_g))
                gds.append(pltpu.async_copy(
                    adstf.at[didxo.at[k]], adst_r.at[j], sem_g))
            for d in gds:
                d.wait()
            for j in range(2):
                for i in range(8):
                    lane = lane0 + i * 16
                    a_s = plsc.load_gather(asrc_r.at[j], [lane, zidx])
                    a_d = plsc.load_gather(adst_r.at[j], [lane, zidx])
                    al = a_s + a_d
                    al = jnp.where(al > 0, al, al * jnp.float32(0.2)) - m
                    plsc.store_scatter(wbuf.at[j], [lane, zidx], jnp.exp(al))

                def scale(e, carry2):
                    ws = plsc.load_gather(
                        wbuf.at[j], [jnp.full((16,), e, jnp.int32), zidx])
                    rows[j, e, pl.ds(0, 16)] = rows[j, e, pl.ds(0, 16)] * ws
                    rows[j, e, pl.ds(16, 16)] = rows[j, e, pl.ds(16, 16)] * ws
                    return carry2

                lax.fori_loop(0, 128, scale, 0, unroll=8)
            sds = []
            for j in range(2):
                k = half * 2 + j
                sds.append(pltpu.async_copy(
                    rows.at[j], accn.at[didx.at[k]], sem_s, add=True))
                sds.append(pltpu.async_copy(
                    wbuf.at[j], wout.at[c * EBP + b0 + k], sem_w))
            for d in sds:
                d.wait()
        return carry

    lax.fori_loop(0, NGRP_SUB, group, 0)
    plsc.subcore_barrier()
    _write_chunk(s, accn, outn.at[c])


def _gatn_call(hflat, asrcf, adstf, mvec, src2d, dst2d, zer32):
    return pl.kernel(
        _gatn_body,
        out_type=[jax.ShapeDtypeStruct((NC, N, 32), jnp.float32),
                  jax.ShapeDtypeStruct((NC * EBP, 128, 8), jnp.float32)],
        mesh=plsc.VectorSubcoreMesh(core_axis_name="c", subcore_axis_name="s",
                                    num_cores=NC, num_subcores=NSUB),
        compiler_params=pltpu.CompilerParams(use_tc_tiling_on_sc=False, needs_layout_passes=False),
        scratch_types=[
            pltpu.VMEM_SHARED((NP8, 32), jnp.float32),
            pltpu.VMEM((GRP, 128), jnp.int32),
            pltpu.VMEM((GRP, 128), jnp.int32),
            pltpu.VMEM((GRP, 128), jnp.int32),
            pltpu.VMEM((GRP, 128), jnp.int32),
            pltpu.VMEM((2, 128, 32), jnp.float32),
            pltpu.VMEM((2, 128, 8), jnp.float32),
            pltpu.VMEM((2, 128, 8), jnp.float32),
            pltpu.VMEM((2, 128, 8), jnp.float32),
            pltpu.VMEM((16,), jnp.float32),
            pltpu.SemaphoreType.DMA,
            pltpu.SemaphoreType.DMA,
            pltpu.SemaphoreType.DMA,
        ],
    )(hflat, asrcf, adstf, mvec, src2d, dst2d, zer32)


def _gatd_body(wtmp, dst2d, zer8, outd, accd, didx, wloc, sem_s):
    c = lax.axis_index("c")
    s = lax.axis_index("s")
    _zero_chunk(s, zer8, accd)
    plsc.subcore_barrier()

    base = s * NGRP_SUB * GRP

    def group(g, carry):
        b0 = base + g * GRP
        pltpu.sync_copy(dst2d.at[pl.ds(b0, GRP)], didx)
        pltpu.sync_copy(wtmp.at[pl.ds(c * EBP + b0, GRP)], wloc)
        sds = []
        for k in range(GRP):
            sds.append(pltpu.async_copy(
                wloc.at[k], accd.at[didx.at[k]], sem_s, add=True))
        for d in sds:
            d.wait()
        return carry

    lax.fori_loop(0, NGRP_SUB, group, 0)
    plsc.subcore_barrier()
    _write_chunk(s, accd, outd.at[c])


def _gatd_call(wtmp, dst2d, zer8):
    return pl.kernel(
        _gatd_body,
        out_type=jax.ShapeDtypeStruct((NC, N, 8), jnp.float32),
        mesh=plsc.VectorSubcoreMesh(core_axis_name="c", subcore_axis_name="s",
                                    num_cores=NC, num_subcores=NSUB),
        compiler_params=pltpu.CompilerParams(use_tc_tiling_on_sc=False, needs_layout_passes=False),
        scratch_types=[
            pltpu.VMEM_SHARED((NP8, 8), jnp.float32),
            pltpu.VMEM((GRP, 128), jnp.int32),
            pltpu.VMEM((GRP, 128, 8), jnp.float32),
            pltpu.SemaphoreType.DMA,
        ],
    )(wtmp, dst2d, zer8)


def _full(shape):
    return pl.BlockSpec(shape, lambda i: tuple(0 for _ in shape))


def _rows(ch):
    return pl.BlockSpec((BR, ch), lambda i: (i, 0))


def _mm(a, b):
    return jnp.dot(a, b, preferred_element_type=jnp.float32)


def _bn(x, g, b):
    return x * (jnp.float32(1.0) / jnp.sqrt(jnp.float32(1.0 + 1e-5))) * g + b


def _k1_body(x, lng, lnb, w1, b1, w2, b2, gw, x2o, h1o):
    xb = x[...]
    mu = jnp.mean(xb, axis=1, keepdims=True)
    xc = xb - mu
    var = jnp.mean(xc * xc, axis=1, keepdims=True)
    ln = xc / jnp.sqrt(var + 1e-5) * lng[...] + lnb[...]
    a = jnp.maximum(_mm(ln, w1[...]) + b1[...], 0.0)
    a = jnp.maximum(_mm(a, w2[...]) + b2[...], 0.0)
    x2o[...] = a
    h1o[...] = _mm(a, gw[...])


def _k2_body(h1, d0, d1, ha, hb, dv):
    deg = d0[...] + d1[...] + 1.0
    dinv = lax.rsqrt(deg)
    h1s = h1[...] * dinv
    ha[...] = h1s[:, :32]
    hb[...] = h1s[:, 32:]
    dv[...] = dinv


def _k3_body(a1a, a1b, hsa, hsb, dv, g1b, bn1g, bn1b, gatw, aw, bw,
             h2a, h2b, s0, s1, t0, t1, mo, msc):
    i = pl.program_id(0)
    dinv = dv[...]
    acc = jnp.concatenate([a1a[...], a1b[...]], axis=1)
    h1s = jnp.concatenate([hsa[...], hsb[...]], axis=1)
    g1 = (acc + h1s) * dinv + g1b[...]
    x3 = jnp.maximum(_bn(g1, bn1g[...], bn1b[...]), 0.0)
    h2 = _mm(x3, gatw[...])
    h2_0, h2_1 = h2[:, :32], h2[:, 32:]
    h2a[...] = h2_0
    h2b[...] = h2_1
    awv, bwv = aw[...], bw[...]
    as0 = jnp.sum(h2_0 * awv[0, :], axis=1, keepdims=True)
    as1 = jnp.sum(h2_1 * awv[1, :], axis=1, keepdims=True)
    ad0 = jnp.sum(h2_0 * bwv[0, :], axis=1, keepdims=True)
    ad1 = jnp.sum(h2_1 * bwv[1, :], axis=1, keepdims=True)
    s0[...] = as0
    s1[...] = as1
    t0[...] = ad0
    t1[...] = ad1

    @pl.when(i == 0)
    def _():
        msc[0] = jnp.float32(-3e38)
        msc[1] = jnp.float32(-3e38)
        msc[2] = jnp.float32(-3e38)
        msc[3] = jnp.float32(-3e38)

    msc[0] = jnp.maximum(msc[0], jnp.max(as0))
    msc[1] = jnp.maximum(msc[1], jnp.max(as1))
    msc[2] = jnp.maximum(msc[2], jnp.max(ad0))
    msc[3] = jnp.maximum(msc[3], jnp.max(ad1))

    @pl.when(i == GRID - 1)
    def _():
        mm_ = jnp.maximum(msc[0] + msc[2], msc[1] + msc[3])
        mm_ = jnp.where(mm_ > 0, mm_, mm_ * jnp.float32(0.2))
        mo[...] = jnp.full((16,), mm_, jnp.float32)


def _k4_body(n0, n1, d0, d1, h2a, h2b, s0, s1, t0, t1, mv, bn2g, bn2b,
             gatb, g2w, dv, h3a, h3b):
    m = mv[0]
    outs = []
    for nh, dh, hh, sh, th in ((n0, d0, h2a, s0, t0), (n1, d1, h2b, s1, t1)):
        al = sh[...] + th[...]
        al = jnp.where(al > 0, al, al * jnp.float32(0.2)) - m
        ws = jnp.exp(al)
        num = nh[...] + ws * hh[...]
        den = dh[...] + ws
        outs.append(num / (den + 1e-16))
    xg = jnp.concatenate(outs, axis=1) + gatb[...]
    x4 = jnp.maximum(_bn(xg, bn2g[...], bn2b[...]), 0.0)
    h3s = _mm(x4, g2w[...]) * dv[...]
    h3a[...] = h3s[:, :16]
    h3b[...] = h3s[:, 16:]


def _k5_body(a2a, a2b, h3a, h3b, dv, g2b, bn3g, bn3b, w1, b1, w2, b2,
             w3, b3, out):
    acc = jnp.concatenate([a2a[...], a2b[...]], axis=1)
    h3s = jnp.concatenate([h3a[...], h3b[...]], axis=1)
    g2 = (acc + h3s) * dv[...] + g2b[...]
    x5 = jnp.maximum(_bn(g2, bn3g[...], bn3b[...]), 0.0)
    p = jnp.maximum(_mm(x5, w1[...]) + b1[...], 0.0)
    p = jnp.maximum(_mm(p, w2[...]) + b2[...], 0.0)
    z = _mm(p, w3[...]) + b3[...]
    sig = jnp.float32(1.0) / (jnp.float32(1.0) + jnp.exp(-z))
    out[...] = jnp.broadcast_to(sig, (BR, 8))


def _flat_halves(ha, hb):
    z = jnp.zeros((8, ha.shape[1]), jnp.float32)
    return jnp.concatenate([ha, z, hb, z], axis=0)


def kernel(accessibility_features, edge_index, ln_g, ln_b, fe_w1, fe_b1,
           fe_w2, fe_b2, gcn1_w, gcn1_b, bn1_g, bn1_b, gat_w, gat_asrc,
           gat_adst, gat_b, bn2_g, bn2_b, gcn2_w, gcn2_b, bn3_g, bn3_b,
           pw1, pb1, pw2, pb2, pw3, pb3):
    f32 = jnp.float32
    pad_s = jnp.full((EPAD,), N, jnp.int32)
    src2d = jnp.concatenate([edge_index[0], pad_s]).reshape(EBP, 128)
    dst2d = jnp.concatenate([edge_index[1], pad_s]).reshape(EBP, 128)
    zer32 = jnp.zeros((ROWC, 32), f32)
    zer16 = jnp.zeros((ROWC, 16), f32)
    zer8d = jnp.zeros((ROWC, 8), f32)
    ones_h = jnp.ones((128, 8), f32)

    degf = _deg_call(dst2d, zer8d, ones_h)

    x2, h1 = pl.pallas_call(
        _k1_body,
        grid=(GRID,),
        in_specs=[_rows(128), _full((128,)), _full((128,)), _full((128, 64)),
                  _full((64,)), _full((64, 64)), _full((64,)),
                  _full((64, 64))],
        out_specs=[_rows(64), _rows(64)],
        out_shape=[jax.ShapeDtypeStruct((N, 64), f32)] * 2,
    )(accessibility_features, ln_g, ln_b, fe_w1, fe_b1, fe_w2, fe_b2, gcn1_w)

    h1s_a, h1s_b, dinv = pl.pallas_call(
        _k2_body,
        grid=(GRID,),
        in_specs=[_rows(64), _rows(1), _rows(1)],
        out_specs=[_rows(32), _rows(32), _rows(1)],
        out_shape=[jax.ShapeDtypeStruct((N, 32), f32),
                   jax.ShapeDtypeStruct((N, 32), f32),
                   jax.ShapeDtypeStruct((N, 1), f32)],
    )(h1, degf[0, :, 0:1], degf[1, :, 0:1])

    acc1 = _gcn_call(32, _flat_halves(h1s_a, h1s_b), src2d, dst2d, zer32)

    h2a, h2b, s0, s1, t0, t1, mvec = pl.pallas_call(
        _k3_body,
        grid=(GRID,),
        in_specs=[_rows(32), _rows(32), _rows(32), _rows(32), _rows(1),
                  _full((64,)), _full((64,)), _full((64,)), _full((64, 64)),
                  _full((2, 32)), _full((2, 32))],
        out_specs=[_rows(32), _rows(32), _rows(1), _rows(1), _rows(1),
                   _rows(1), pl.BlockSpec((16,), lambda i: (0,))],
        out_shape=[jax.ShapeDtypeStruct((N, 32), f32),
                   jax.ShapeDtypeStruct((N, 32), f32),
                   jax.ShapeDtypeStruct((N, 1), f32),
                   jax.ShapeDtypeStruct((N, 1), f32),
                   jax.ShapeDtypeStruct((N, 1), f32),
                   jax.ShapeDtypeStruct((N, 1), f32),
                   jax.ShapeDtypeStruct((16,), f32)],
        scratch_shapes=[pltpu.SMEM((4,), f32)],
    )(acc1[0], acc1[1], h1s_a, h1s_b, dinv, gcn1_b, bn1_g, bn1_b, gat_w,
      gat_asrc, gat_adst)

    zp8 = jnp.zeros((8, 8), f32)
    pad7 = ((0, 0), (0, 7))
    asrcf = jnp.concatenate([jnp.pad(s0, pad7), zp8,
                             jnp.pad(s1, pad7), zp8], axis=0)
    adstf = jnp.concatenate([jnp.pad(t0, pad7), zp8,
                             jnp.pad(t1, pad7), zp8], axis=0)
    zer8 = jnp.zeros((ROWC, 8), f32)
    h2flat = _flat_halves(h2a, h2b)
    outn, wtmp = _gatn_call(h2flat, asrcf, adstf, mvec, src2d, dst2d, zer32)
    outd = _gatd_call(wtmp, dst2d, zer8)

    h3a, h3b = pl.pallas_call(
        _k4_body,
        grid=(GRID,),
        in_specs=[_rows(32), _rows(32), _rows(1), _rows(1), _rows(32),
                  _rows(32), _rows(1), _rows(1), _rows(1), _rows(1),
                  _full((16,)), _full((64,)), _full((64,)), _full((64,)),
                  _full((64, 32)), _rows(1)],
        out_specs=[_rows(16), _rows(16)],
        out_shape=[jax.ShapeDtypeStruct((N, 16), f32)] * 2,
    )(outn[0], outn[1], outd[0, :, 0:1], outd[1, :, 0:1],
      h2a, h2b, s0, s1, t0, t1, mvec, bn2_g, bn2_b, gat_b, gcn2_w, dinv)

    acc2 = _gcn_call(16, _flat_halves(h3a, h3b), src2d, dst2d, zer16)

    out = pl.pallas_call(
        _k5_body,
        grid=(GRID,),
        in_specs=[_rows(16), _rows(16), _rows(16), _rows(16), _rows(1),
                  _full((32,)), _full((32,)), _full((32,)),
                  _full((32, 16)), _full((16,)), _full((16, 8)),
                  _full((8,)), _full((8, 1)), _full((1,))],
        out_specs=[_rows(8)],
        out_shape=[jax.ShapeDtypeStruct((N, 8), f32)],
    )(acc2[0], acc2[1], h3a, h3b, dinv, gcn2_b, bn3_g, bn3_b, pw1, pb1,
      pw2, pb2, pw3, pb3)[0]

    return out[:, 0]
